# baseline (speedup 1.0000x reference)
import functools

import jax
import jax.numpy as jnp
from jax.experimental import pallas as pl
from jax.experimental.pallas import tpu as pltpu

J = 21
H = 4
HID = 64
ROT = 6

_PARENT_SLICES = ((0, 1), (0, 4), (0, 1), (5, 8), (0, 1), (9, 12),
                  (0, 1), (13, 16), (0, 1), (17, 20))


def _parent_gather(t):
    return jnp.concatenate([t[a:b] for a, b in _PARENT_SLICES], axis=0)


def _leaky(v):
    return jnp.maximum(v, 0.2 * v)


def _gat_layer(x, W, WaT, mask, blk):
    k_in = x.shape[-1]
    xf = x.reshape(J * blk, k_in)
    hf = jnp.dot(xf, W, preferred_element_type=jnp.float32)
    h3 = hf.reshape(J, blk, H * HID)
    hpar = _parent_gather(h3)
    sc = jax.lax.dot_general(WaT, xf, (((1,), (1,)), ((), ())),
                             preferred_element_type=jnp.float32)
    sc4 = sc.reshape(2 * H, J, blk)
    a_src = sc4[:H]
    a_dst = sc4[H:]
    as_par = jnp.concatenate([a_src[:, a:b] for a, b in _PARENT_SLICES],
                             axis=1)
    e_self = _leaky(a_src + a_dst)
    e_par = jnp.where(mask, _leaky(as_par + a_dst), -1e30)
    al = jax.nn.sigmoid(e_self - e_par)
    alb = al.astype(jnp.bfloat16).reshape(H, J * blk)
    lane = jax.lax.broadcasted_iota(jnp.int32, (H, H * HID), 1) // HID
    head = jax.lax.broadcasted_iota(jnp.int32, (H, H * HID), 0)
    R = (lane == head).astype(jnp.bfloat16)
    alf = jax.lax.dot_general(alb, R, (((0,), (0,)), ((), ())),
                              preferred_element_type=jnp.float32)
    alf = alf.reshape(J, blk, H * HID)
    return alf * (h3 - hpar) + hpar


def _body(xj_ref, gr_ref, W1_ref, Wa1_ref, b1_ref,
          W2_ref, Wa2_ref, b2_ref,
          W3_ref, Wa3_ref,
          Wtr_ref, btp_ref, Wg1r_ref, bg1p_ref, Wg2_ref, bg2_ref,
          rot_ref, g_ref, *, blk):
    mask = jax.lax.broadcasted_iota(jnp.int32, (H, J, blk), 1) > 0

    gr = gr_ref[...]
    x = jnp.concatenate(
        [xj_ref[...], jnp.broadcast_to(gr[None], (J, blk, ROT))], axis=-1)
    x = x.astype(jnp.bfloat16)

    out = _gat_layer(x, W1_ref[...], Wa1_ref[...], mask, blk)
    x1 = jax.nn.relu(out + b1_ref[...]).astype(jnp.bfloat16)

    out = _gat_layer(x1, W2_ref[...], Wa2_ref[...], mask, blk)
    x2 = jax.nn.relu(out + b2_ref[...]).astype(jnp.bfloat16)

    out3 = _gat_layer(x2, W3_ref[...], Wa3_ref[...], mask, blk)
    o3b = out3.astype(jnp.bfloat16)

    rot = jnp.dot(o3b.reshape(J * blk, H * HID), Wtr_ref[...],
                  preferred_element_type=jnp.float32) + btp_ref[...]
    rot_ref[...] = rot.reshape(J, blk, ROT)

    s = jnp.sum(out3, axis=0).astype(jnp.bfloat16)
    g1 = jax.nn.relu(jnp.dot(s, Wg1r_ref[...],
                             preferred_element_type=jnp.float32)
                     + bg1p_ref[...])
    g_ref[...] = jnp.dot(g1.astype(jnp.bfloat16), Wg2_ref[...],
                         preferred_element_type=jnp.float32) + bg2_ref[...]


def kernel(joints, global_rotation, W1, a1s, a1d, b1, W2, a2s, a2d, b2,
           W3, a3s, a3d, b3, Wt, bt, Wg1, bg1, Wg2, bg2,
           edge_index, batch):
    B = joints.shape[0]
    blk = 256
    grid = B // blk

    xj = joints.transpose(1, 0, 2)

    eye = jnp.eye(H, dtype=jnp.float32)

    def att_w(W, a_s, a_d):
        As = (a_s[:, :, None] * eye[:, None, :]).reshape(H * HID, H)
        Ad = (a_d[:, :, None] * eye[:, None, :]).reshape(H * HID, H)
        return (W @ jnp.concatenate([As, Ad], axis=1)).T.astype(jnp.bfloat16)

    Wa1 = att_w(W1, a1s, a1d)
    Wa2 = att_w(W2, a2s, a2d)
    Wa3 = att_w(W3, a3s, a3d)
    W1b = W1.astype(jnp.bfloat16)
    W2b = W2.astype(jnp.bfloat16)
    W3b = W3.astype(jnp.bfloat16)

    Wtr = (jnp.concatenate([Wt] * H, axis=0) * 0.25).astype(jnp.bfloat16)
    btp = (b3 @ Wt + bt).reshape(1, ROT)
    Wg1r = (jnp.concatenate([Wg1] * H, axis=0)
            * (1.0 / (H * J))).astype(jnp.bfloat16)
    bg1p = (b3 @ Wg1 + bg1).reshape(1, HID)
    Wg2b = Wg2.astype(jnp.bfloat16)

    b1r = b1.reshape(1, H * HID)
    b2r = b2.reshape(1, H * HID)
    bg2r = bg2.reshape(1, ROT)

    const = lambda *shape: pl.BlockSpec(shape, lambda i: (0,) * len(shape))

    rot_t, g = pl.pallas_call(
        functools.partial(_body, blk=blk),
        grid=(grid,),
        compiler_params=pltpu.CompilerParams(
            dimension_semantics=("parallel",)),
        in_specs=[
            pl.BlockSpec((J, blk, 3), lambda i: (0, i, 0)),
            pl.BlockSpec((blk, ROT), lambda i: (i, 0)),
            const(3 + ROT, H * HID), const(2 * H, 3 + ROT),
            const(1, H * HID),
            const(H * HID, H * HID), const(2 * H, H * HID),
            const(1, H * HID),
            const(H * HID, H * HID), const(2 * H, H * HID),
            const(H * HID, ROT), const(1, ROT),
            const(H * HID, HID), const(1, HID),
            const(HID, ROT), const(1, ROT),
        ],
        out_specs=[
            pl.BlockSpec((J, blk, ROT), lambda i: (0, i, 0)),
            pl.BlockSpec((blk, ROT), lambda i: (i, 0)),
        ],
        out_shape=[
            jax.ShapeDtypeStruct((J, B, ROT), jnp.float32),
            jax.ShapeDtypeStruct((B, ROT), jnp.float32),
        ],
    )(xj, global_rotation, W1b, Wa1, b1r, W2b, Wa2, b2r,
      W3b, Wa3, Wtr, btp, Wg1r, bg1p, Wg2b, bg2r)

    return rot_t.transpose(1, 0, 2), g


# speedup vs baseline: 1.0143x; 1.0143x over previous
import functools

import jax
import jax.numpy as jnp
from jax.experimental import pallas as pl
from jax.experimental.pallas import tpu as pltpu

J = 21
H = 4
HID = 64
ROT = 6

_PARENT_SLICES = ((0, 1), (0, 4), (0, 1), (5, 8), (0, 1), (9, 12),
                  (0, 1), (13, 16), (0, 1), (17, 20))


def _parent_gather(t):
    return jnp.concatenate([t[a:b] for a, b in _PARENT_SLICES], axis=0)


def _leaky(v):
    return jnp.maximum(v, 0.2 * v)


def _gat_layer(x, W, WaT, mask, blk):
    k_in = x.shape[-1]
    xf = x.reshape(J * blk, k_in)
    hf = jnp.dot(xf, W,
                 preferred_element_type=jnp.float32).astype(jnp.bfloat16)
    h3 = hf.reshape(J, blk, H * HID)
    hpar = _parent_gather(h3)
    sc = jax.lax.dot_general(WaT, xf, (((1,), (1,)), ((), ())),
                             preferred_element_type=jnp.float32)
    sc4 = sc.reshape(2 * H, J, blk)
    a_src = sc4[:H]
    a_dst = sc4[H:]
    as_par = jnp.concatenate([a_src[:, a:b] for a, b in _PARENT_SLICES],
                             axis=1)
    e_self = _leaky(a_src + a_dst)
    e_par = jnp.where(mask, _leaky(as_par + a_dst), -1e30)
    al = jax.nn.sigmoid(e_self - e_par)
    alb = al.astype(jnp.bfloat16).reshape(H, J * blk)
    lane = jax.lax.broadcasted_iota(jnp.int32, (H, H * HID), 1) // HID
    head = jax.lax.broadcasted_iota(jnp.int32, (H, H * HID), 0)
    R = (lane == head).astype(jnp.bfloat16)
    alf = jax.lax.dot_general(alb, R, (((0,), (0,)), ((), ())),
                              preferred_element_type=jnp.float32)
    alf = alf.astype(jnp.bfloat16).reshape(J, blk, H * HID)
    return alf * (h3 - hpar) + hpar


def _body(xj_ref, gr_ref, W1_ref, Wa1_ref, b1_ref,
          W2_ref, Wa2_ref, b2_ref,
          W3_ref, Wa3_ref,
          Wtr_ref, btp_ref, Wg1r_ref, bg1p_ref, Wg2_ref, bg2_ref,
          rot_ref, g_ref, *, blk):
    mask = jax.lax.broadcasted_iota(jnp.int32, (H, J, blk), 1) > 0

    gr = gr_ref[...]
    x = jnp.concatenate(
        [xj_ref[...], jnp.broadcast_to(gr[None], (J, blk, ROT))], axis=-1)
    x = x.astype(jnp.bfloat16)

    out = _gat_layer(x, W1_ref[...], Wa1_ref[...], mask, blk)
    x1 = jax.nn.relu(out + b1_ref[...])

    out = _gat_layer(x1, W2_ref[...], Wa2_ref[...], mask, blk)
    x2 = jax.nn.relu(out + b2_ref[...])

    out3 = _gat_layer(x2, W3_ref[...], Wa3_ref[...], mask, blk)

    rot = jnp.dot(out3.reshape(J * blk, H * HID), Wtr_ref[...],
                  preferred_element_type=jnp.float32) + btp_ref[...]
    rot_ref[...] = rot.reshape(J, blk, ROT)

    s = jnp.sum(out3.astype(jnp.float32), axis=0).astype(jnp.bfloat16)
    g1 = jax.nn.relu(jnp.dot(s, Wg1r_ref[...],
                             preferred_element_type=jnp.float32)
                     + bg1p_ref[...])
    g_ref[...] = jnp.dot(g1.astype(jnp.bfloat16), Wg2_ref[...],
                         preferred_element_type=jnp.float32) + bg2_ref[...]


def kernel(joints, global_rotation, W1, a1s, a1d, b1, W2, a2s, a2d, b2,
           W3, a3s, a3d, b3, Wt, bt, Wg1, bg1, Wg2, bg2,
           edge_index, batch):
    B = joints.shape[0]
    blk = 256
    grid = B // blk

    xj = joints.transpose(1, 0, 2)

    eye = jnp.eye(H, dtype=jnp.float32)

    def att_w(W, a_s, a_d):
        As = (a_s[:, :, None] * eye[:, None, :]).reshape(H * HID, H)
        Ad = (a_d[:, :, None] * eye[:, None, :]).reshape(H * HID, H)
        return (W @ jnp.concatenate([As, Ad], axis=1)).T.astype(jnp.bfloat16)

    Wa1 = att_w(W1, a1s, a1d)
    Wa2 = att_w(W2, a2s, a2d)
    Wa3 = att_w(W3, a3s, a3d)
    W1b = W1.astype(jnp.bfloat16)
    W2b = W2.astype(jnp.bfloat16)
    W3b = W3.astype(jnp.bfloat16)

    Wtr = (jnp.concatenate([Wt] * H, axis=0) * 0.25).astype(jnp.bfloat16)
    btp = (b3 @ Wt + bt).reshape(1, ROT)
    Wg1r = (jnp.concatenate([Wg1] * H, axis=0)
            * (1.0 / (H * J))).astype(jnp.bfloat16)
    bg1p = (b3 @ Wg1 + bg1).reshape(1, HID)
    Wg2b = Wg2.astype(jnp.bfloat16)

    b1r = b1.reshape(1, H * HID).astype(jnp.bfloat16)
    b2r = b2.reshape(1, H * HID).astype(jnp.bfloat16)
    bg2r = bg2.reshape(1, ROT)

    const = lambda *shape: pl.BlockSpec(shape, lambda i: (0,) * len(shape))

    rot_t, g = pl.pallas_call(
        functools.partial(_body, blk=blk),
        grid=(grid,),
        in_specs=[
            pl.BlockSpec((J, blk, 3), lambda i: (0, i, 0)),
            pl.BlockSpec((blk, ROT), lambda i: (i, 0)),
            const(3 + ROT, H * HID), const(2 * H, 3 + ROT),
            const(1, H * HID),
            const(H * HID, H * HID), const(2 * H, H * HID),
            const(1, H * HID),
            const(H * HID, H * HID), const(2 * H, H * HID),
            const(H * HID, ROT), const(1, ROT),
            const(H * HID, HID), const(1, HID),
            const(HID, ROT), const(1, ROT),
        ],
        out_specs=[
            pl.BlockSpec((J, blk, ROT), lambda i: (0, i, 0)),
            pl.BlockSpec((blk, ROT), lambda i: (i, 0)),
        ],
        out_shape=[
            jax.ShapeDtypeStruct((J, B, ROT), jnp.float32),
            jax.ShapeDtypeStruct((B, ROT), jnp.float32),
        ],
    )(xj, global_rotation, W1b, Wa1, b1r, W2b, Wa2, b2r,
      W3b, Wa3, Wtr, btp, Wg1r, bg1p, Wg2b, bg2r)

    return rot_t.transpose(1, 0, 2), g


# speedup vs baseline: 1.0640x; 1.0491x over previous
import functools

import jax
import jax.numpy as jnp
from jax.experimental import pallas as pl
from jax.experimental.pallas import tpu as pltpu

J = 21
H = 4
HID = 64
ROT = 6

_PARENT_SLICES = ((0, 1), (0, 4), (0, 1), (5, 8), (0, 1), (9, 12),
                  (0, 1), (13, 16), (0, 1), (17, 20))


def _parent_gather(t):
    return jnp.concatenate([t[a:b] for a, b in _PARENT_SLICES], axis=0)


def _leaky(v):
    return jnp.maximum(v, 0.2 * v)


def _gat_layer(x, W, WaT, mask, blk):
    k_in = x.shape[-1]
    xf = x.reshape(J * blk, k_in)
    hf = jnp.dot(xf, W,
                 preferred_element_type=jnp.float32).astype(jnp.bfloat16)
    h3 = hf.reshape(J, blk, H * HID)
    hpar = _parent_gather(h3)
    sc = jax.lax.dot_general(WaT, xf, (((1,), (1,)), ((), ())),
                             preferred_element_type=jnp.float32)
    sc4 = sc.reshape(2 * H, J, blk)
    a_src = sc4[:H]
    a_dst = sc4[H:]
    as_par = jnp.concatenate([a_src[:, a:b] for a, b in _PARENT_SLICES],
                             axis=1)
    e_self = _leaky(a_src + a_dst)
    e_par = jnp.where(mask, _leaky(as_par + a_dst), -1e30)
    al = jax.nn.sigmoid(e_self - e_par)
    alb = al.astype(jnp.bfloat16).reshape(H, J * blk)
    lane = jax.lax.broadcasted_iota(jnp.int32, (H, H * HID), 1) // HID
    head = jax.lax.broadcasted_iota(jnp.int32, (H, H * HID), 0)
    R = (lane == head).astype(jnp.bfloat16)
    alf = jax.lax.dot_general(alb, R, (((0,), (0,)), ((), ())),
                              preferred_element_type=jnp.float32)
    alf = alf.astype(jnp.bfloat16).reshape(J, blk, H * HID)
    return alf * (h3 - hpar) + hpar


def _body(xj_ref, gr_ref, W1_ref, Wa1_ref, b1_ref,
          W2_ref, Wa2_ref, b2_ref,
          W3_ref, Wa3_ref,
          Wtr_ref, btp_ref, Wg1r_ref, bg1p_ref, Wg2_ref, bg2_ref,
          rot_ref, g_ref, *, blk):
    mask = jax.lax.broadcasted_iota(jnp.int32, (H, J, blk), 1) > 0

    gr = gr_ref[...]
    x = jnp.concatenate(
        [xj_ref[...], jnp.broadcast_to(gr[None], (J, blk, ROT))], axis=-1)
    x = x.astype(jnp.bfloat16)

    out = _gat_layer(x, W1_ref[...], Wa1_ref[...], mask, blk)
    x1 = jax.nn.relu(out + b1_ref[...])

    out = _gat_layer(x1, W2_ref[...], Wa2_ref[...], mask, blk)
    x2 = jax.nn.relu(out + b2_ref[...])

    out3 = _gat_layer(x2, W3_ref[...], Wa3_ref[...], mask, blk)

    rot = jnp.dot(out3.reshape(J * blk, H * HID), Wtr_ref[...],
                  preferred_element_type=jnp.float32) + btp_ref[...]
    rot_ref[...] = rot.reshape(J, blk, ROT)

    s = jnp.sum(out3.astype(jnp.float32), axis=0).astype(jnp.bfloat16)
    g1 = jax.nn.relu(jnp.dot(s, Wg1r_ref[...],
                             preferred_element_type=jnp.float32)
                     + bg1p_ref[...])
    g_ref[...] = jnp.dot(g1.astype(jnp.bfloat16), Wg2_ref[...],
                         preferred_element_type=jnp.float32) + bg2_ref[...]


def kernel(joints, global_rotation, W1, a1s, a1d, b1, W2, a2s, a2d, b2,
           W3, a3s, a3d, b3, Wt, bt, Wg1, bg1, Wg2, bg2,
           edge_index, batch):
    B = joints.shape[0]
    blk = 512
    grid = B // blk

    xj = joints.transpose(1, 0, 2)

    eye = jnp.eye(H, dtype=jnp.float32)

    def att_w(W, a_s, a_d):
        As = (a_s[:, :, None] * eye[:, None, :]).reshape(H * HID, H)
        Ad = (a_d[:, :, None] * eye[:, None, :]).reshape(H * HID, H)
        return (W @ jnp.concatenate([As, Ad], axis=1)).T.astype(jnp.bfloat16)

    Wa1 = att_w(W1, a1s, a1d)
    Wa2 = att_w(W2, a2s, a2d)
    Wa3 = att_w(W3, a3s, a3d)
    W1b = W1.astype(jnp.bfloat16)
    W2b = W2.astype(jnp.bfloat16)
    W3b = W3.astype(jnp.bfloat16)

    Wtr = (jnp.concatenate([Wt] * H, axis=0) * 0.25).astype(jnp.bfloat16)
    btp = (b3 @ Wt + bt).reshape(1, ROT)
    Wg1r = (jnp.concatenate([Wg1] * H, axis=0)
            * (1.0 / (H * J))).astype(jnp.bfloat16)
    bg1p = (b3 @ Wg1 + bg1).reshape(1, HID)
    Wg2b = Wg2.astype(jnp.bfloat16)

    b1r = b1.reshape(1, H * HID).astype(jnp.bfloat16)
    b2r = b2.reshape(1, H * HID).astype(jnp.bfloat16)
    bg2r = bg2.reshape(1, ROT)

    const = lambda *shape: pl.BlockSpec(shape, lambda i: (0,) * len(shape))

    rot_t, g = pl.pallas_call(
        functools.partial(_body, blk=blk),
        grid=(grid,),
        in_specs=[
            pl.BlockSpec((J, blk, 3), lambda i: (0, i, 0)),
            pl.BlockSpec((blk, ROT), lambda i: (i, 0)),
            const(3 + ROT, H * HID), const(2 * H, 3 + ROT),
            const(1, H * HID),
            const(H * HID, H * HID), const(2 * H, H * HID),
            const(1, H * HID),
            const(H * HID, H * HID), const(2 * H, H * HID),
            const(H * HID, ROT), const(1, ROT),
            const(H * HID, HID), const(1, HID),
            const(HID, ROT), const(1, ROT),
        ],
        out_specs=[
            pl.BlockSpec((J, blk, ROT), lambda i: (0, i, 0)),
            pl.BlockSpec((blk, ROT), lambda i: (i, 0)),
        ],
        out_shape=[
            jax.ShapeDtypeStruct((J, B, ROT), jnp.float32),
            jax.ShapeDtypeStruct((B, ROT), jnp.float32),
        ],
    )(xj, global_rotation, W1b, Wa1, b1r, W2b, Wa2, b2r,
      W3b, Wa3, Wtr, btp, Wg1r, bg1p, Wg2b, bg2r)

    return rot_t.transpose(1, 0, 2), g


# speedup vs baseline: 1.4026x; 1.3182x over previous
import functools

import jax
import jax.numpy as jnp
from jax.experimental import pallas as pl
from jax.experimental.pallas import tpu as pltpu

J = 21
H = 4
HID = 64
ROT = 6

_PARENT_SLICES = ((0, 1), (0, 4), (0, 1), (5, 8), (0, 1), (9, 12),
                  (0, 1), (13, 16), (0, 1), (17, 20))


def _parent_gather(t):
    return jnp.concatenate([t[a:b] for a, b in _PARENT_SLICES], axis=0)


def _leaky(v):
    return jnp.maximum(v, 0.2 * v)


def _gat_layer(x, W, WaT, mask, blk, transposed_in=False):
    if transposed_in:
        hf = jax.lax.dot_general(x, W, (((0,), (0,)), ((), ())),
                                 preferred_element_type=jnp.float32)
        sc = jax.lax.dot_general(WaT, x, (((1,), (0,)), ((), ())),
                                 preferred_element_type=jnp.float32)
    else:
        xf = x.reshape(J * blk, x.shape[-1])
        hf = jnp.dot(xf, W, preferred_element_type=jnp.float32)
        sc = jax.lax.dot_general(WaT, xf, (((1,), (1,)), ((), ())),
                                 preferred_element_type=jnp.float32)
    hf = hf.astype(jnp.bfloat16)
    h3 = hf.reshape(J, blk, H * HID)
    hpar = _parent_gather(h3)
    sc4 = sc.reshape(2 * H, J, blk)
    a_src = sc4[:H]
    a_dst = sc4[H:]
    as_par = jnp.concatenate([a_src[:, a:b] for a, b in _PARENT_SLICES],
                             axis=1)
    e_self = _leaky(a_src + a_dst)
    e_par = jnp.where(mask, _leaky(as_par + a_dst), -1e30)
    al = jax.nn.sigmoid(e_self - e_par)
    alb = al.astype(jnp.bfloat16).reshape(H, J * blk)
    lane = jax.lax.broadcasted_iota(jnp.int32, (H, H * HID), 1) // HID
    head = jax.lax.broadcasted_iota(jnp.int32, (H, H * HID), 0)
    R = (lane == head).astype(jnp.bfloat16)
    alf = jax.lax.dot_general(alb, R, (((0,), (0,)), ((), ())),
                              preferred_element_type=jnp.float32)
    alf = alf.astype(jnp.bfloat16).reshape(J, blk, H * HID)
    return alf * (h3 - hpar) + hpar


def _body(jt_ref, gr_ref, W1_ref, Wa1_ref, b1_ref,
          W2_ref, Wa2_ref, b2_ref,
          W3_ref, Wa3_ref,
          WtrT_ref, btpT_ref, Wg1r_ref, bg1p_ref, Wg2T_ref, bg2T_ref,
          rot_ref, g_ref, *, blk):
    mask = jax.lax.broadcasted_iota(jnp.int32, (H, J, blk), 1) > 0

    g6 = jnp.broadcast_to(gr_ref[...][:, None, :], (ROT, J, blk))
    xT = jnp.concatenate([jt_ref[...], g6], axis=0)
    xT = xT.astype(jnp.bfloat16).reshape(3 + ROT, J * blk)

    out = _gat_layer(xT, W1_ref[...], Wa1_ref[...], mask, blk,
                     transposed_in=True)
    x1 = jax.nn.relu(out + b1_ref[...])

    out = _gat_layer(x1, W2_ref[...], Wa2_ref[...], mask, blk)
    x2 = jax.nn.relu(out + b2_ref[...])

    out3 = _gat_layer(x2, W3_ref[...], Wa3_ref[...], mask, blk)
    o3f = out3.reshape(J * blk, H * HID)

    rotT = jax.lax.dot_general(WtrT_ref[...], o3f, (((1,), (1,)), ((), ())),
                               preferred_element_type=jnp.float32)
    rot_ref[...] = (rotT + btpT_ref[...]).reshape(ROT, J, blk)

    s = jnp.sum(out3.astype(jnp.float32), axis=0).astype(jnp.bfloat16)
    g1 = jax.nn.relu(jnp.dot(s, Wg1r_ref[...],
                             preferred_element_type=jnp.float32)
                     + bg1p_ref[...]).astype(jnp.bfloat16)
    gT = jax.lax.dot_general(Wg2T_ref[...], g1, (((1,), (1,)), ((), ())),
                             preferred_element_type=jnp.float32)
    g_ref[...] = gT + bg2T_ref[...]


def kernel(joints, global_rotation, W1, a1s, a1d, b1, W2, a2s, a2d, b2,
           W3, a3s, a3d, b3, Wt, bt, Wg1, bg1, Wg2, bg2,
           edge_index, batch):
    B = joints.shape[0]
    blk = 512
    grid = B // blk

    jt = joints.transpose(2, 1, 0)
    grT = global_rotation.T

    eye = jnp.eye(H, dtype=jnp.float32)

    def att_w(W, a_s, a_d):
        As = (a_s[:, :, None] * eye[:, None, :]).reshape(H * HID, H)
        Ad = (a_d[:, :, None] * eye[:, None, :]).reshape(H * HID, H)
        return (W @ jnp.concatenate([As, Ad], axis=1)).T.astype(jnp.bfloat16)

    Wa1 = att_w(W1, a1s, a1d)
    Wa2 = att_w(W2, a2s, a2d)
    Wa3 = att_w(W3, a3s, a3d)
    W1b = W1.astype(jnp.bfloat16)
    W2b = W2.astype(jnp.bfloat16)
    W3b = W3.astype(jnp.bfloat16)

    WtrT = (jnp.concatenate([Wt] * H, axis=0).T * 0.25).astype(jnp.bfloat16)
    btpT = (b3 @ Wt + bt).reshape(ROT, 1)
    Wg1r = (jnp.concatenate([Wg1] * H, axis=0)
            * (1.0 / (H * J))).astype(jnp.bfloat16)
    bg1p = (b3 @ Wg1 + bg1).reshape(1, HID)
    Wg2T = Wg2.T.astype(jnp.bfloat16)
    bg2T = bg2.reshape(ROT, 1)

    b1r = b1.reshape(1, H * HID).astype(jnp.bfloat16)
    b2r = b2.reshape(1, H * HID).astype(jnp.bfloat16)

    const = lambda *shape: pl.BlockSpec(shape, lambda i: (0,) * len(shape))

    rot_t, g_t = pl.pallas_call(
        functools.partial(_body, blk=blk),
        grid=(grid,),
        in_specs=[
            pl.BlockSpec((3, J, blk), lambda i: (0, 0, i)),
            pl.BlockSpec((ROT, blk), lambda i: (0, i)),
            const(3 + ROT, H * HID), const(2 * H, 3 + ROT),
            const(1, H * HID),
            const(H * HID, H * HID), const(2 * H, H * HID),
            const(1, H * HID),
            const(H * HID, H * HID), const(2 * H, H * HID),
            const(ROT, H * HID), const(ROT, 1),
            const(H * HID, HID), const(1, HID),
            const(ROT, HID), const(ROT, 1),
        ],
        out_specs=[
            pl.BlockSpec((ROT, J, blk), lambda i: (0, 0, i)),
            pl.BlockSpec((ROT, blk), lambda i: (0, i)),
        ],
        out_shape=[
            jax.ShapeDtypeStruct((ROT, J, B), jnp.float32),
            jax.ShapeDtypeStruct((ROT, B), jnp.float32),
        ],
    )(jt, grT, W1b, Wa1, b1r, W2b, Wa2, b2r,
      W3b, Wa3, WtrT, btpT, Wg1r, bg1p, Wg2T, bg2T)

    return rot_t.transpose(2, 1, 0), g_t.T
